# Initial kernel scaffold; baseline (speedup 1.0000x reference)
import jax
import jax.numpy as jnp
from jax.experimental import pallas as pl


def kernel(x, edge_index, batch, w1, b1, bn_g, bn_b, bn_rm, bn_rv, w2, b2, wp, bp, ln_g, ln_b):
    raise NotImplementedError("write your pallas kernel here")


# speedup vs baseline: 3.1039x; 3.1039x over previous
import functools

import jax
import jax.numpy as jnp
from jax import lax
from jax.experimental import pallas as pl
from jax.experimental.pallas import tpu as pltpu
from jax.experimental.pallas import tpu_sc as plsc

N = 10000
E = 320000
D = 128
G = 256
DEPTH = 3

NC = 2
NS = 16
NW = NC * NS
NP = 10240
CL = 128
NCH = 80
RPT = NP // NS
RCH = RPT // CL

BLK = 400
NBLK = N // BLK


def _sc_scatter_add(h, src3, dst3):
    mesh = plsc.VectorSubcoreMesh(core_axis_name="c", subcore_axis_name="s")

    @functools.partial(
        pl.kernel,
        out_type=jax.ShapeDtypeStruct((NC, NP, D), jnp.float32),
        mesh=mesh,
        scratch_types=[
            pltpu.VMEM((NCH // 2, CL), jnp.int32),
            pltpu.VMEM((NCH // 2, CL), jnp.int32),
            pltpu.VMEM((CL, D), jnp.float32),
            pltpu.VMEM((CL, D), jnp.float32),
            pltpu.VMEM_SHARED((NP, D), jnp.float32),
            pltpu.SemaphoreType.DMA,
            pltpu.SemaphoreType.DMA,
        ],
    )
    def k(h_hbm, src_hbm, dst_hbm, out_hbm, src_v, dst_v, buf_a, buf_b, acc,
          sem_a, sem_b):
        cid = lax.axis_index("c")
        sid = lax.axis_index("s")
        wid = sid * NC + cid
        half = NCH // 2

        def zbody(i, c):
            r = i >> 3
            col = (i & 7) << 4
            buf_a[r, pl.ds(col, 16)] = jnp.zeros((16,), jnp.float32)
            return c

        lax.fori_loop(0, CL * (D // 16), zbody, 0)
        for r in range(RCH):
            pltpu.sync_copy(buf_a, acc.at[pl.ds(sid * RPT + r * CL, CL)])
        plsc.subcore_barrier()

        def ebody(i, c):
            g0 = 2 * i
            g1 = g0 + 1
            pltpu.async_copy(h_hbm.at[src_v.at[g1]], buf_b, sem_b)
            pltpu.make_async_copy(h_hbm.at[src_v.at[g0]], buf_a, sem_a).wait()
            pltpu.sync_copy(buf_a, acc.at[dst_v.at[g0]], add=True)

            @pl.when(g0 + 2 < half)
            def _():
                pltpu.async_copy(h_hbm.at[src_v.at[g0 + 2]], buf_a, sem_a)

            pltpu.make_async_copy(h_hbm.at[src_v.at[g1]], buf_b, sem_b).wait()
            pltpu.sync_copy(buf_b, acc.at[dst_v.at[g1]], add=True)
            return c

        for hf in range(2):
            pltpu.sync_copy(src_hbm.at[wid, pl.ds(hf * half, half)], src_v)
            pltpu.sync_copy(dst_hbm.at[wid, pl.ds(hf * half, half)], dst_v)
            pltpu.async_copy(h_hbm.at[src_v.at[0]], buf_a, sem_a)
            lax.fori_loop(0, half // 2, ebody, 0)

        plsc.subcore_barrier()
        for r in range(RCH):
            off = sid * RPT + r * CL
            pltpu.sync_copy(acc.at[pl.ds(off, CL)], buf_a)
            pltpu.sync_copy(buf_a, out_hbm.at[cid, pl.ds(off, CL)])

    return k(h, src3, dst3)


def _mlp_body(h_ref, a0_ref, a1_ref, w1_ref, b1_ref, w2_ref, b2_ref, o_ref):
    z = h_ref[...] + a0_ref[0] + a1_ref[0]
    z = jnp.maximum(
        jnp.dot(z, w1_ref[...], preferred_element_type=jnp.float32)
        + b1_ref[...], 0.0)
    o_ref[...] = jnp.maximum(
        jnp.dot(z, w2_ref[...], preferred_element_type=jnp.float32)
        + b2_ref[...], 0.0)


def _mlp(h, agg, w1f, b1f, w2, b2):
    return pl.pallas_call(
        _mlp_body,
        grid=(NBLK,),
        in_specs=[
            pl.BlockSpec((BLK, D), lambda i: (i, 0)),
            pl.BlockSpec((1, BLK, D), lambda i: (0, i, 0)),
            pl.BlockSpec((1, BLK, D), lambda i: (1, i, 0)),
            pl.BlockSpec((D, D), lambda i: (0, 0)),
            pl.BlockSpec((1, D), lambda i: (0, 0)),
            pl.BlockSpec((D, D), lambda i: (0, 0)),
            pl.BlockSpec((1, D), lambda i: (0, 0)),
        ],
        out_specs=pl.BlockSpec((BLK, D), lambda i: (i, 0)),
        out_shape=jax.ShapeDtypeStruct((N, D), jnp.float32),
    )(h, agg, agg, w1f, b1f, w2, b2)


def _mlp_pool_body(h_ref, a0_ref, a1_ref, w1_ref, b1_ref, w2_ref, b2_ref,
                   bt_ref, wp_ref, bp_ref, lg_ref, lb_ref, y_ref, seg, cnt):
    i = pl.program_id(0)

    @pl.when(i == 0)
    def _():
        seg[...] = jnp.zeros_like(seg)
        cnt[...] = jnp.zeros_like(cnt)

    z = h_ref[...] + a0_ref[0] + a1_ref[0]
    z = jnp.maximum(
        jnp.dot(z, w1_ref[...], preferred_element_type=jnp.float32)
        + b1_ref[...], 0.0)
    o = jnp.maximum(
        jnp.dot(z, w2_ref[...], preferred_element_type=jnp.float32)
        + b2_ref[...], 0.0)

    b = bt_ref[0, 0, :]
    seg_ids = lax.broadcasted_iota(jnp.int32, (G, BLK), 0)
    pf = (seg_ids == b[None, :]).astype(jnp.float32)
    seg[...] += jnp.dot(pf, o, preferred_element_type=jnp.float32)
    cnt[...] += jnp.broadcast_to(
        jnp.sum(pf, axis=1, keepdims=True), (G, D))

    @pl.when(i == NBLK - 1)
    def _():
        mean = seg[...] / jnp.maximum(cnt[...], 1.0)
        y = jnp.dot(mean, wp_ref[...],
                    preferred_element_type=jnp.float32) + bp_ref[...]
        mu = jnp.mean(y, axis=-1, keepdims=True)
        var = jnp.mean((y - mu) ** 2, axis=-1, keepdims=True)
        y_ref[...] = (y - mu) * lax.rsqrt(var + 1e-5) * lg_ref[...] + lb_ref[...]


def _mlp_pool(h, agg, w1f, b1f, w2, b2, batch3, wp, bp, ln_g, ln_b):
    return pl.pallas_call(
        _mlp_pool_body,
        grid=(NBLK,),
        in_specs=[
            pl.BlockSpec((BLK, D), lambda i: (i, 0)),
            pl.BlockSpec((1, BLK, D), lambda i: (0, i, 0)),
            pl.BlockSpec((1, BLK, D), lambda i: (1, i, 0)),
            pl.BlockSpec((D, D), lambda i: (0, 0)),
            pl.BlockSpec((1, D), lambda i: (0, 0)),
            pl.BlockSpec((D, D), lambda i: (0, 0)),
            pl.BlockSpec((1, D), lambda i: (0, 0)),
            pl.BlockSpec((1, 1, BLK), lambda i: (i, 0, 0)),
            pl.BlockSpec((D, D), lambda i: (0, 0)),
            pl.BlockSpec((1, D), lambda i: (0, 0)),
            pl.BlockSpec((1, D), lambda i: (0, 0)),
            pl.BlockSpec((1, D), lambda i: (0, 0)),
        ],
        out_specs=pl.BlockSpec((G, D), lambda i: (0, 0)),
        out_shape=jax.ShapeDtypeStruct((G, D), jnp.float32),
        scratch_shapes=[
            pltpu.VMEM((G, D), jnp.float32),
            pltpu.VMEM((G, D), jnp.float32),
        ],
    )(h, agg, agg, w1f, b1f, w2, b2, batch3, wp, bp, ln_g, ln_b)


def kernel(x, edge_index, batch, w1, b1, bn_g, bn_b, bn_rm, bn_rv, w2, b2,
           wp, bp, ln_g, ln_b):
    scale = bn_g / jnp.sqrt(bn_rv + 1e-5)
    w1f = w1 * scale[:, None, :]
    b1f = (b1 - bn_rm) * scale + bn_b

    pad = NW * NCH * CL - E
    src3 = jnp.concatenate(
        [edge_index[0], jnp.zeros((pad,), jnp.int32)]).reshape(NW, NCH, CL)
    dst3 = jnp.concatenate(
        [edge_index[1], jnp.full((pad,), NP - 1, jnp.int32)]).reshape(
            NW, NCH, CL)
    batch3 = batch.reshape(NBLK, 1, BLK)

    h = x
    for i in range(DEPTH - 1):
        agg = _sc_scatter_add(h, src3, dst3)
        h = _mlp(h, agg, w1f[i], b1f[i][None], w2[i], b2[i][None])
    agg = _sc_scatter_add(h, src3, dst3)
    return _mlp_pool(h, agg, w1f[2], b1f[2][None], w2[2], b2[2][None],
                     batch3, wp, bp[None], ln_g[None], ln_b[None])


# speedup vs baseline: 8.5510x; 2.7549x over previous
import functools

import jax
import jax.numpy as jnp
from jax import lax
from jax.experimental import pallas as pl
from jax.experimental.pallas import tpu as pltpu
from jax.experimental.pallas import tpu_sc as plsc

N = 10000
E = 320000
D = 128
G = 256
DEPTH = 3

NC = 2
NS = 16
NW = NC * NS
NP = 10240
CL = 128
NCH = 80
RPT = NP // NS
RCH = RPT // CL

BLK = 400
NBLK = N // BLK


def _sc_scatter_add(h, src3, dst3):
    mesh = plsc.VectorSubcoreMesh(core_axis_name="c", subcore_axis_name="s")

    @functools.partial(
        pl.kernel,
        out_type=jax.ShapeDtypeStruct((NC, NP, D), jnp.float32),
        mesh=mesh,
        scratch_types=[
            pltpu.VMEM((NCH // 2, CL), jnp.int32),
            pltpu.VMEM((NCH // 2, CL), jnp.int32),
            pltpu.VMEM((CL, D), jnp.float32),
            pltpu.VMEM((CL, D), jnp.float32),
            pltpu.VMEM_SHARED((NP, D), jnp.float32),
            pltpu.SemaphoreType.DMA,
            pltpu.SemaphoreType.DMA,
            pltpu.SemaphoreType.DMA,
            pltpu.SemaphoreType.DMA,
        ],
    )
    def k(h_hbm, src_hbm, dst_hbm, out_hbm, src_v, dst_v, buf_a, buf_b, acc,
          sem_a, sem_b, sem_sa, sem_sb):
        cid = lax.axis_index("c")
        sid = lax.axis_index("s")
        wid = sid * NC + cid
        half = NCH // 2

        def zbody(i, c):
            r = i >> 3
            col = (i & 7) << 4
            buf_a[r, pl.ds(col, 16)] = jnp.zeros((16,), jnp.float32)
            return c

        lax.fori_loop(0, CL * (D // 16), zbody, 0)
        for r in range(RCH):
            pltpu.sync_copy(buf_a, acc.at[pl.ds(sid * RPT + r * CL, CL)])
        plsc.subcore_barrier()

        def ebody(i, c):
            g0 = 2 * i
            g1 = g0 + 1
            pltpu.make_async_copy(h_hbm.at[src_v.at[g0]], buf_a, sem_a).wait()
            pltpu.async_copy(buf_a, acc.at[dst_v.at[g0]], sem_sa, add=True)
            pltpu.make_async_copy(h_hbm.at[src_v.at[g1]], buf_b, sem_b).wait()
            pltpu.async_copy(buf_b, acc.at[dst_v.at[g1]], sem_sb, add=True)

            pltpu.make_async_copy(buf_a, acc.at[dst_v.at[g0]], sem_sa).wait()

            @pl.when(g0 + 2 < half)
            def _():
                pltpu.async_copy(h_hbm.at[src_v.at[g0 + 2]], buf_a, sem_a)

            pltpu.make_async_copy(buf_b, acc.at[dst_v.at[g1]], sem_sb).wait()

            @pl.when(g1 + 2 < half)
            def _():
                pltpu.async_copy(h_hbm.at[src_v.at[g1 + 2]], buf_b, sem_b)

            return c

        for hf in range(2):
            pltpu.sync_copy(src_hbm.at[wid, pl.ds(hf * half, half)], src_v)
            pltpu.sync_copy(dst_hbm.at[wid, pl.ds(hf * half, half)], dst_v)
            pltpu.async_copy(h_hbm.at[src_v.at[0]], buf_a, sem_a)
            pltpu.async_copy(h_hbm.at[src_v.at[1]], buf_b, sem_b)
            lax.fori_loop(0, half // 2, ebody, 0)

        plsc.subcore_barrier()
        for r in range(RCH):
            off = sid * RPT + r * CL
            pltpu.sync_copy(acc.at[pl.ds(off, CL)], buf_a)
            pltpu.sync_copy(buf_a, out_hbm.at[cid, pl.ds(off, CL)])

    return k(h, src3, dst3)


def _mlp_body(h_ref, a0_ref, a1_ref, w1_ref, b1_ref, w2_ref, b2_ref, o_ref):
    z = h_ref[...] + a0_ref[0] + a1_ref[0]
    z = jnp.maximum(
        jnp.dot(z, w1_ref[...], preferred_element_type=jnp.float32)
        + b1_ref[...], 0.0)
    o_ref[...] = jnp.maximum(
        jnp.dot(z, w2_ref[...], preferred_element_type=jnp.float32)
        + b2_ref[...], 0.0)


def _mlp(h, agg, w1f, b1f, w2, b2):
    return pl.pallas_call(
        _mlp_body,
        grid=(NBLK,),
        in_specs=[
            pl.BlockSpec((BLK, D), lambda i: (i, 0)),
            pl.BlockSpec((1, BLK, D), lambda i: (0, i, 0)),
            pl.BlockSpec((1, BLK, D), lambda i: (1, i, 0)),
            pl.BlockSpec((D, D), lambda i: (0, 0)),
            pl.BlockSpec((1, D), lambda i: (0, 0)),
            pl.BlockSpec((D, D), lambda i: (0, 0)),
            pl.BlockSpec((1, D), lambda i: (0, 0)),
        ],
        out_specs=pl.BlockSpec((BLK, D), lambda i: (i, 0)),
        out_shape=jax.ShapeDtypeStruct((N, D), jnp.float32),
    )(h, agg, agg, w1f, b1f, w2, b2)


def _mlp_pool_body(h_ref, a0_ref, a1_ref, w1_ref, b1_ref, w2_ref, b2_ref,
                   bt_ref, wp_ref, bp_ref, lg_ref, lb_ref, y_ref, seg, cnt):
    i = pl.program_id(0)

    @pl.when(i == 0)
    def _():
        seg[...] = jnp.zeros_like(seg)
        cnt[...] = jnp.zeros_like(cnt)

    z = h_ref[...] + a0_ref[0] + a1_ref[0]
    z = jnp.maximum(
        jnp.dot(z, w1_ref[...], preferred_element_type=jnp.float32)
        + b1_ref[...], 0.0)
    o = jnp.maximum(
        jnp.dot(z, w2_ref[...], preferred_element_type=jnp.float32)
        + b2_ref[...], 0.0)

    b = bt_ref[0, 0, :]
    seg_ids = lax.broadcasted_iota(jnp.int32, (G, BLK), 0)
    pf = (seg_ids == b[None, :]).astype(jnp.float32)
    seg[...] += jnp.dot(pf, o, preferred_element_type=jnp.float32)
    cnt[...] += jnp.broadcast_to(
        jnp.sum(pf, axis=1, keepdims=True), (G, D))

    @pl.when(i == NBLK - 1)
    def _():
        mean = seg[...] / jnp.maximum(cnt[...], 1.0)
        y = jnp.dot(mean, wp_ref[...],
                    preferred_element_type=jnp.float32) + bp_ref[...]
        mu = jnp.mean(y, axis=-1, keepdims=True)
        var = jnp.mean((y - mu) ** 2, axis=-1, keepdims=True)
        y_ref[...] = (y - mu) * lax.rsqrt(var + 1e-5) * lg_ref[...] + lb_ref[...]


def _mlp_pool(h, agg, w1f, b1f, w2, b2, batch3, wp, bp, ln_g, ln_b):
    return pl.pallas_call(
        _mlp_pool_body,
        grid=(NBLK,),
        in_specs=[
            pl.BlockSpec((BLK, D), lambda i: (i, 0)),
            pl.BlockSpec((1, BLK, D), lambda i: (0, i, 0)),
            pl.BlockSpec((1, BLK, D), lambda i: (1, i, 0)),
            pl.BlockSpec((D, D), lambda i: (0, 0)),
            pl.BlockSpec((1, D), lambda i: (0, 0)),
            pl.BlockSpec((D, D), lambda i: (0, 0)),
            pl.BlockSpec((1, D), lambda i: (0, 0)),
            pl.BlockSpec((1, 1, BLK), lambda i: (i, 0, 0)),
            pl.BlockSpec((D, D), lambda i: (0, 0)),
            pl.BlockSpec((1, D), lambda i: (0, 0)),
            pl.BlockSpec((1, D), lambda i: (0, 0)),
            pl.BlockSpec((1, D), lambda i: (0, 0)),
        ],
        out_specs=pl.BlockSpec((G, D), lambda i: (0, 0)),
        out_shape=jax.ShapeDtypeStruct((G, D), jnp.float32),
        scratch_shapes=[
            pltpu.VMEM((G, D), jnp.float32),
            pltpu.VMEM((G, D), jnp.float32),
        ],
    )(h, agg, agg, w1f, b1f, w2, b2, batch3, wp, bp, ln_g, ln_b)


def kernel(x, edge_index, batch, w1, b1, bn_g, bn_b, bn_rm, bn_rv, w2, b2,
           wp, bp, ln_g, ln_b):
    scale = bn_g / jnp.sqrt(bn_rv + 1e-5)
    w1f = w1 * scale[:, None, :]
    b1f = (b1 - bn_rm) * scale + bn_b

    pad = NW * NCH * CL - E
    pad_ar = jnp.arange(pad, dtype=jnp.int32)
    src3 = jnp.concatenate(
        [edge_index[0], pad_ar % N]).reshape(NW, NCH, CL)
    dst3 = jnp.concatenate(
        [edge_index[1], N + pad_ar % (NP - N)]).reshape(NW, NCH, CL)
    batch3 = batch.reshape(NBLK, 1, BLK)

    h = x
    for i in range(DEPTH - 1):
        agg = _sc_scatter_add(h, src3, dst3)
        h = _mlp(h, agg, w1f[i], b1f[i][None], w2[i], b2[i][None])
    agg = _sc_scatter_add(h, src3, dst3)
    return _mlp_pool(h, agg, w1f[2], b1f[2][None], w2[2], b2[2][None],
                     batch3, wp, bp[None], ln_g[None], ln_b[None])


# speedup vs baseline: 8.7158x; 1.0193x over previous
import functools

import jax
import jax.numpy as jnp
from jax import lax
from jax.experimental import pallas as pl
from jax.experimental.pallas import tpu as pltpu
from jax.experimental.pallas import tpu_sc as plsc

N = 10000
E = 320000
D = 128
G = 256
DEPTH = 3

NC = 2
NS = 16
NW = NC * NS
NP = 10240
CL = 128
NCH = 80
RPT = NP // NS
RCH = RPT // CL

BLK = 400
NBLK = N // BLK


def _sc_scatter_add(h, src3, dst3):
    mesh = plsc.VectorSubcoreMesh(core_axis_name="c", subcore_axis_name="s")

    @functools.partial(
        pl.kernel,
        out_type=jax.ShapeDtypeStruct((NC, NP, D), jnp.float32),
        mesh=mesh,
        scratch_types=[
            pltpu.VMEM((NCH // 2, CL), jnp.int32),
            pltpu.VMEM((NCH // 2, CL), jnp.int32),
            pltpu.VMEM((CL, D), jnp.float32),
            pltpu.VMEM((CL, D), jnp.float32),
            pltpu.VMEM_SHARED((NP, D), jnp.float32),
            pltpu.SemaphoreType.DMA,
            pltpu.SemaphoreType.DMA,
            pltpu.SemaphoreType.DMA,
            pltpu.SemaphoreType.DMA,
        ],
    )
    def k(h_hbm, src_hbm, dst_hbm, out_hbm, src_v, dst_v, buf_a, buf_b, acc,
          sem_a, sem_b, sem_sa, sem_sb):
        cid = lax.axis_index("c")
        sid = lax.axis_index("s")
        wid = sid * NC + cid
        half = NCH // 2

        pltpu.async_copy(src_hbm.at[wid, pl.ds(0, half)], src_v, sem_a)
        pltpu.async_copy(dst_hbm.at[wid, pl.ds(0, half)], dst_v, sem_b)

        def zbody(i, c):
            r = i >> 3
            col = (i & 7) << 4
            buf_a[r, pl.ds(col, 16)] = jnp.zeros((16,), jnp.float32)
            return c

        lax.fori_loop(0, CL * (D // 16), zbody, 0)
        for r in range(RCH):
            off = pl.multiple_of(sid * RPT + r * CL, CL)
            pltpu.async_copy(buf_a, acc.at[pl.ds(off, CL)], sem_sa)
        for r in range(RCH):
            off = pl.multiple_of(sid * RPT + r * CL, CL)
            pltpu.make_async_copy(buf_a, acc.at[pl.ds(off, CL)], sem_sa).wait()
        pltpu.make_async_copy(src_hbm.at[wid, pl.ds(0, half)], src_v,
                              sem_a).wait()
        pltpu.make_async_copy(dst_hbm.at[wid, pl.ds(0, half)], dst_v,
                              sem_b).wait()
        pltpu.async_copy(h_hbm.at[src_v.at[0]], buf_a, sem_a)
        pltpu.async_copy(h_hbm.at[src_v.at[1]], buf_b, sem_b)
        plsc.subcore_barrier()

        def ebody(i, c):
            g0 = 2 * i
            g1 = g0 + 1
            pltpu.make_async_copy(h_hbm.at[src_v.at[g0]], buf_a, sem_a).wait()
            pltpu.async_copy(buf_a, acc.at[dst_v.at[g0]], sem_sa, add=True)
            pltpu.make_async_copy(h_hbm.at[src_v.at[g1]], buf_b, sem_b).wait()
            pltpu.async_copy(buf_b, acc.at[dst_v.at[g1]], sem_sb, add=True)

            pltpu.make_async_copy(buf_a, acc.at[dst_v.at[g0]], sem_sa).wait()

            @pl.when(g0 + 2 < half)
            def _():
                pltpu.async_copy(h_hbm.at[src_v.at[g0 + 2]], buf_a, sem_a)

            pltpu.make_async_copy(buf_b, acc.at[dst_v.at[g1]], sem_sb).wait()

            @pl.when(g1 + 2 < half)
            def _():
                pltpu.async_copy(h_hbm.at[src_v.at[g1 + 2]], buf_b, sem_b)

            return c

        lax.fori_loop(0, half // 2, ebody, 0)

        pltpu.sync_copy(src_hbm.at[wid, pl.ds(half, half)], src_v)
        pltpu.sync_copy(dst_hbm.at[wid, pl.ds(half, half)], dst_v)
        pltpu.async_copy(h_hbm.at[src_v.at[0]], buf_a, sem_a)
        pltpu.async_copy(h_hbm.at[src_v.at[1]], buf_b, sem_b)
        lax.fori_loop(0, half // 2, ebody, 0)

        plsc.subcore_barrier()
        offs = [pl.multiple_of(sid * RPT + r * CL, CL) for r in range(RCH)]
        bufs = [buf_a if r % 2 == 0 else buf_b for r in range(RCH)]
        isem = [sem_a if r % 2 == 0 else sem_b for r in range(RCH)]
        osem = [sem_sa if r % 2 == 0 else sem_sb for r in range(RCH)]
        pltpu.async_copy(acc.at[pl.ds(offs[0], CL)], bufs[0], isem[0])
        for r in range(RCH):
            pltpu.make_async_copy(acc.at[pl.ds(offs[r], CL)], bufs[r],
                                  isem[r]).wait()
            if r >= 1:
                pltpu.make_async_copy(bufs[r - 1],
                                      out_hbm.at[cid, pl.ds(offs[r - 1], CL)],
                                      osem[r - 1]).wait()
            pltpu.async_copy(bufs[r], out_hbm.at[cid, pl.ds(offs[r], CL)],
                             osem[r])
            if r + 1 < RCH:
                pltpu.async_copy(acc.at[pl.ds(offs[r + 1], CL)], bufs[r + 1],
                                 isem[r + 1])
        pltpu.make_async_copy(bufs[RCH - 1],
                              out_hbm.at[cid, pl.ds(offs[RCH - 1], CL)],
                              osem[RCH - 1]).wait()

    return k(h, src3, dst3)


def _mlp_body(h_ref, a0_ref, a1_ref, w1_ref, b1_ref, w2_ref, b2_ref, o_ref):
    z = h_ref[...] + a0_ref[0] + a1_ref[0]
    z = jnp.maximum(
        jnp.dot(z, w1_ref[...], preferred_element_type=jnp.float32)
        + b1_ref[...], 0.0)
    o_ref[...] = jnp.maximum(
        jnp.dot(z, w2_ref[...], preferred_element_type=jnp.float32)
        + b2_ref[...], 0.0)


def _mlp(h, agg, w1f, b1f, w2, b2):
    return pl.pallas_call(
        _mlp_body,
        grid=(NBLK,),
        in_specs=[
            pl.BlockSpec((BLK, D), lambda i: (i, 0)),
            pl.BlockSpec((1, BLK, D), lambda i: (0, i, 0)),
            pl.BlockSpec((1, BLK, D), lambda i: (1, i, 0)),
            pl.BlockSpec((D, D), lambda i: (0, 0)),
            pl.BlockSpec((1, D), lambda i: (0, 0)),
            pl.BlockSpec((D, D), lambda i: (0, 0)),
            pl.BlockSpec((1, D), lambda i: (0, 0)),
        ],
        out_specs=pl.BlockSpec((BLK, D), lambda i: (i, 0)),
        out_shape=jax.ShapeDtypeStruct((N, D), jnp.float32),
    )(h, agg, agg, w1f, b1f, w2, b2)


def _mlp_pool_body(h_ref, a0_ref, a1_ref, w1_ref, b1_ref, w2_ref, b2_ref,
                   bt_ref, wp_ref, bp_ref, lg_ref, lb_ref, y_ref, seg, cnt):
    i = pl.program_id(0)

    @pl.when(i == 0)
    def _():
        seg[...] = jnp.zeros_like(seg)
        cnt[...] = jnp.zeros_like(cnt)

    z = h_ref[...] + a0_ref[0] + a1_ref[0]
    z = jnp.maximum(
        jnp.dot(z, w1_ref[...], preferred_element_type=jnp.float32)
        + b1_ref[...], 0.0)
    o = jnp.maximum(
        jnp.dot(z, w2_ref[...], preferred_element_type=jnp.float32)
        + b2_ref[...], 0.0)

    b = bt_ref[0, 0, :]
    seg_ids = lax.broadcasted_iota(jnp.int32, (G, BLK), 0)
    pf = (seg_ids == b[None, :]).astype(jnp.float32)
    seg[...] += jnp.dot(pf, o, preferred_element_type=jnp.float32)
    cnt[...] += jnp.broadcast_to(
        jnp.sum(pf, axis=1, keepdims=True), (G, D))

    @pl.when(i == NBLK - 1)
    def _():
        mean = seg[...] / jnp.maximum(cnt[...], 1.0)
        y = jnp.dot(mean, wp_ref[...],
                    preferred_element_type=jnp.float32) + bp_ref[...]
        mu = jnp.mean(y, axis=-1, keepdims=True)
        var = jnp.mean((y - mu) ** 2, axis=-1, keepdims=True)
        y_ref[...] = (y - mu) * lax.rsqrt(var + 1e-5) * lg_ref[...] + lb_ref[...]


def _mlp_pool(h, agg, w1f, b1f, w2, b2, batch3, wp, bp, ln_g, ln_b):
    return pl.pallas_call(
        _mlp_pool_body,
        grid=(NBLK,),
        in_specs=[
            pl.BlockSpec((BLK, D), lambda i: (i, 0)),
            pl.BlockSpec((1, BLK, D), lambda i: (0, i, 0)),
            pl.BlockSpec((1, BLK, D), lambda i: (1, i, 0)),
            pl.BlockSpec((D, D), lambda i: (0, 0)),
            pl.BlockSpec((1, D), lambda i: (0, 0)),
            pl.BlockSpec((D, D), lambda i: (0, 0)),
            pl.BlockSpec((1, D), lambda i: (0, 0)),
            pl.BlockSpec((1, 1, BLK), lambda i: (i, 0, 0)),
            pl.BlockSpec((D, D), lambda i: (0, 0)),
            pl.BlockSpec((1, D), lambda i: (0, 0)),
            pl.BlockSpec((1, D), lambda i: (0, 0)),
            pl.BlockSpec((1, D), lambda i: (0, 0)),
        ],
        out_specs=pl.BlockSpec((G, D), lambda i: (0, 0)),
        out_shape=jax.ShapeDtypeStruct((G, D), jnp.float32),
        scratch_shapes=[
            pltpu.VMEM((G, D), jnp.float32),
            pltpu.VMEM((G, D), jnp.float32),
        ],
    )(h, agg, agg, w1f, b1f, w2, b2, batch3, wp, bp, ln_g, ln_b)


def kernel(x, edge_index, batch, w1, b1, bn_g, bn_b, bn_rm, bn_rv, w2, b2,
           wp, bp, ln_g, ln_b):
    scale = bn_g / jnp.sqrt(bn_rv + 1e-5)
    w1f = w1 * scale[:, None, :]
    b1f = (b1 - bn_rm) * scale + bn_b

    pad = NW * NCH * CL - E
    pad_ar = jnp.arange(pad, dtype=jnp.int32)
    src3 = jnp.concatenate(
        [edge_index[0], pad_ar % N]).reshape(NW, NCH, CL)
    dst3 = jnp.concatenate(
        [edge_index[1], N + pad_ar % (NP - N)]).reshape(NW, NCH, CL)
    batch3 = batch.reshape(NBLK, 1, BLK)

    h = x
    for i in range(DEPTH - 1):
        agg = _sc_scatter_add(h, src3, dst3)
        h = _mlp(h, agg, w1f[i], b1f[i][None], w2[i], b2[i][None])
    agg = _sc_scatter_add(h, src3, dst3)
    return _mlp_pool(h, agg, w1f[2], b1f[2][None], w2[2], b2[2][None],
                     batch3, wp, bp[None], ln_g[None], ln_b[None])


# speedup vs baseline: 9.4885x; 1.0887x over previous
import functools

import jax
import jax.numpy as jnp
from jax import lax
from jax.experimental import pallas as pl
from jax.experimental.pallas import tpu as pltpu
from jax.experimental.pallas import tpu_sc as plsc

N = 10000
E = 320000
D = 128
G = 256
DEPTH = 3

NC = 2
NS = 16
NW = NC * NS
NP = 10240
CL = 128
NCH = 80
RPT = NP // NS
RCH = RPT // CL

BLK = 400
NBLK = N // BLK


def _sc_scatter_add(h, src3, dst3):
    mesh = plsc.VectorSubcoreMesh(core_axis_name="c", subcore_axis_name="s")

    @functools.partial(
        pl.kernel,
        out_type=jax.ShapeDtypeStruct((NC, NP, D), jnp.float32),
        mesh=mesh,
        scratch_types=[
            pltpu.VMEM((NCH // 2, CL), jnp.int32),
            pltpu.VMEM((NCH // 2, CL), jnp.int32),
            pltpu.VMEM((CL, D), jnp.float32),
            pltpu.VMEM((CL, D), jnp.float32),
            pltpu.VMEM_SHARED((NP, D), jnp.float32),
            pltpu.SemaphoreType.DMA,
            pltpu.SemaphoreType.DMA,
            pltpu.SemaphoreType.DMA,
            pltpu.SemaphoreType.DMA,
        ],
    )
    def k(h_hbm, src_hbm, dst_hbm, out_hbm, src_v, dst_v, buf_a, buf_b, acc,
          sem_a, sem_b, sem_sa, sem_sb):
        cid = lax.axis_index("c")
        sid = lax.axis_index("s")
        wid = sid * NC + cid
        half = NCH // 2

        pltpu.async_copy(src_hbm.at[wid, pl.ds(0, half)], src_v, sem_a)
        pltpu.async_copy(dst_hbm.at[wid, pl.ds(0, half)], dst_v, sem_b)

        def zbody(i, c):
            r = i >> 3
            col = (i & 7) << 4
            buf_a[r, pl.ds(col, 16)] = jnp.zeros((16,), jnp.float32)
            return c

        lax.fori_loop(0, CL * (D // 16), zbody, 0)
        for r in range(RCH):
            off = pl.multiple_of(sid * RPT + r * CL, CL)
            pltpu.async_copy(buf_a, acc.at[pl.ds(off, CL)], sem_sa)
        for r in range(RCH):
            off = pl.multiple_of(sid * RPT + r * CL, CL)
            pltpu.make_async_copy(buf_a, acc.at[pl.ds(off, CL)], sem_sa).wait()
        pltpu.make_async_copy(src_hbm.at[wid, pl.ds(0, half)], src_v,
                              sem_a).wait()
        pltpu.make_async_copy(dst_hbm.at[wid, pl.ds(0, half)], dst_v,
                              sem_b).wait()
        pltpu.async_copy(h_hbm.at[src_v.at[0]], buf_a, sem_a)
        plsc.subcore_barrier()

        def ebody(i, c):
            g0 = 2 * i
            g1 = g0 + 1
            pltpu.make_async_copy(h_hbm.at[src_v.at[g0]], buf_a, sem_a).wait()
            pltpu.async_copy(buf_a, acc.at[dst_v.at[g0]], sem_sa, add=True)

            @pl.when(g0 > 0)
            def _():
                pltpu.make_async_copy(buf_b, acc.at[dst_v.at[g0 - 1]],
                                      sem_sb).wait()

            pltpu.async_copy(h_hbm.at[src_v.at[g1]], buf_b, sem_b)

            pltpu.make_async_copy(h_hbm.at[src_v.at[g1]], buf_b, sem_b).wait()
            pltpu.async_copy(buf_b, acc.at[dst_v.at[g1]], sem_sb, add=True)
            pltpu.make_async_copy(buf_a, acc.at[dst_v.at[g0]], sem_sa).wait()

            @pl.when(g1 + 1 < half)
            def _():
                pltpu.async_copy(h_hbm.at[src_v.at[g1 + 1]], buf_a, sem_a)

            return c

        lax.fori_loop(0, half // 2, ebody, 0)
        pltpu.make_async_copy(buf_b, acc.at[dst_v.at[half - 1]],
                              sem_sb).wait()

        pltpu.sync_copy(src_hbm.at[wid, pl.ds(half, half)], src_v)
        pltpu.sync_copy(dst_hbm.at[wid, pl.ds(half, half)], dst_v)
        pltpu.async_copy(h_hbm.at[src_v.at[0]], buf_a, sem_a)
        lax.fori_loop(0, half // 2, ebody, 0)
        pltpu.make_async_copy(buf_b, acc.at[dst_v.at[half - 1]],
                              sem_sb).wait()

        plsc.subcore_barrier()
        offs = [pl.multiple_of(sid * RPT + r * CL, CL) for r in range(RCH)]
        bufs = [buf_a if r % 2 == 0 else buf_b for r in range(RCH)]
        isem = [sem_a if r % 2 == 0 else sem_b for r in range(RCH)]
        osem = [sem_sa if r % 2 == 0 else sem_sb for r in range(RCH)]
        pltpu.async_copy(acc.at[pl.ds(offs[0], CL)], bufs[0], isem[0])
        for r in range(RCH):
            pltpu.make_async_copy(acc.at[pl.ds(offs[r], CL)], bufs[r],
                                  isem[r]).wait()
            if r >= 1:
                pltpu.make_async_copy(bufs[r - 1],
                                      out_hbm.at[cid, pl.ds(offs[r - 1], CL)],
                                      osem[r - 1]).wait()
            pltpu.async_copy(bufs[r], out_hbm.at[cid, pl.ds(offs[r], CL)],
                             osem[r])
            if r + 1 < RCH:
                pltpu.async_copy(acc.at[pl.ds(offs[r + 1], CL)], bufs[r + 1],
                                 isem[r + 1])
        pltpu.make_async_copy(bufs[RCH - 1],
                              out_hbm.at[cid, pl.ds(offs[RCH - 1], CL)],
                              osem[RCH - 1]).wait()

    return k(h, src3, dst3)


def _mlp_body(h_ref, a0_ref, a1_ref, w1_ref, b1_ref, w2_ref, b2_ref, o_ref):
    z = h_ref[...] + a0_ref[0] + a1_ref[0]
    z = jnp.maximum(
        jnp.dot(z, w1_ref[...], preferred_element_type=jnp.float32)
        + b1_ref[...], 0.0)
    o_ref[...] = jnp.maximum(
        jnp.dot(z, w2_ref[...], preferred_element_type=jnp.float32)
        + b2_ref[...], 0.0)


def _mlp(h, agg, w1f, b1f, w2, b2):
    return pl.pallas_call(
        _mlp_body,
        grid=(NBLK,),
        in_specs=[
            pl.BlockSpec((BLK, D), lambda i: (i, 0)),
            pl.BlockSpec((1, BLK, D), lambda i: (0, i, 0)),
            pl.BlockSpec((1, BLK, D), lambda i: (1, i, 0)),
            pl.BlockSpec((D, D), lambda i: (0, 0)),
            pl.BlockSpec((1, D), lambda i: (0, 0)),
            pl.BlockSpec((D, D), lambda i: (0, 0)),
            pl.BlockSpec((1, D), lambda i: (0, 0)),
        ],
        out_specs=pl.BlockSpec((BLK, D), lambda i: (i, 0)),
        out_shape=jax.ShapeDtypeStruct((N, D), jnp.float32),
    )(h, agg, agg, w1f, b1f, w2, b2)


def _mlp_pool_body(h_ref, a0_ref, a1_ref, w1_ref, b1_ref, w2_ref, b2_ref,
                   bt_ref, wp_ref, bp_ref, lg_ref, lb_ref, y_ref, seg, cnt):
    i = pl.program_id(0)

    @pl.when(i == 0)
    def _():
        seg[...] = jnp.zeros_like(seg)
        cnt[...] = jnp.zeros_like(cnt)

    z = h_ref[...] + a0_ref[0] + a1_ref[0]
    z = jnp.maximum(
        jnp.dot(z, w1_ref[...], preferred_element_type=jnp.float32)
        + b1_ref[...], 0.0)
    o = jnp.maximum(
        jnp.dot(z, w2_ref[...], preferred_element_type=jnp.float32)
        + b2_ref[...], 0.0)

    b = bt_ref[0, 0, :]
    seg_ids = lax.broadcasted_iota(jnp.int32, (G, BLK), 0)
    pf = (seg_ids == b[None, :]).astype(jnp.float32)
    seg[...] += jnp.dot(pf, o, preferred_element_type=jnp.float32)
    cnt[...] += jnp.broadcast_to(
        jnp.sum(pf, axis=1, keepdims=True), (G, D))

    @pl.when(i == NBLK - 1)
    def _():
        mean = seg[...] / jnp.maximum(cnt[...], 1.0)
        y = jnp.dot(mean, wp_ref[...],
                    preferred_element_type=jnp.float32) + bp_ref[...]
        mu = jnp.mean(y, axis=-1, keepdims=True)
        var = jnp.mean((y - mu) ** 2, axis=-1, keepdims=True)
        y_ref[...] = (y - mu) * lax.rsqrt(var + 1e-5) * lg_ref[...] + lb_ref[...]


def _mlp_pool(h, agg, w1f, b1f, w2, b2, batch3, wp, bp, ln_g, ln_b):
    return pl.pallas_call(
        _mlp_pool_body,
        grid=(NBLK,),
        in_specs=[
            pl.BlockSpec((BLK, D), lambda i: (i, 0)),
            pl.BlockSpec((1, BLK, D), lambda i: (0, i, 0)),
            pl.BlockSpec((1, BLK, D), lambda i: (1, i, 0)),
            pl.BlockSpec((D, D), lambda i: (0, 0)),
            pl.BlockSpec((1, D), lambda i: (0, 0)),
            pl.BlockSpec((D, D), lambda i: (0, 0)),
            pl.BlockSpec((1, D), lambda i: (0, 0)),
            pl.BlockSpec((1, 1, BLK), lambda i: (i, 0, 0)),
            pl.BlockSpec((D, D), lambda i: (0, 0)),
            pl.BlockSpec((1, D), lambda i: (0, 0)),
            pl.BlockSpec((1, D), lambda i: (0, 0)),
            pl.BlockSpec((1, D), lambda i: (0, 0)),
        ],
        out_specs=pl.BlockSpec((G, D), lambda i: (0, 0)),
        out_shape=jax.ShapeDtypeStruct((G, D), jnp.float32),
        scratch_shapes=[
            pltpu.VMEM((G, D), jnp.float32),
            pltpu.VMEM((G, D), jnp.float32),
        ],
    )(h, agg, agg, w1f, b1f, w2, b2, batch3, wp, bp, ln_g, ln_b)


def kernel(x, edge_index, batch, w1, b1, bn_g, bn_b, bn_rm, bn_rv, w2, b2,
           wp, bp, ln_g, ln_b):
    scale = bn_g / jnp.sqrt(bn_rv + 1e-5)
    w1f = w1 * scale[:, None, :]
    b1f = (b1 - bn_rm) * scale + bn_b

    pad = NW * NCH * CL - E
    pad_ar = jnp.arange(pad, dtype=jnp.int32)
    src3 = jnp.concatenate(
        [edge_index[0], pad_ar % N]).reshape(NW, NCH, CL)
    dst3 = jnp.concatenate(
        [edge_index[1], N + pad_ar % (NP - N)]).reshape(NW, NCH, CL)
    batch3 = batch.reshape(NBLK, 1, BLK)

    h = x
    for i in range(DEPTH - 1):
        agg = _sc_scatter_add(h, src3, dst3)
        h = _mlp(h, agg, w1f[i], b1f[i][None], w2[i], b2[i][None])
    agg = _sc_scatter_add(h, src3, dst3)
    return _mlp_pool(h, agg, w1f[2], b1f[2][None], w2[2], b2[2][None],
                     batch3, wp, bp[None], ln_g[None], ln_b[None])


# speedup vs baseline: 9.5667x; 1.0082x over previous
import functools

import jax
import jax.numpy as jnp
from jax import lax
from jax.experimental import pallas as pl
from jax.experimental.pallas import tpu as pltpu
from jax.experimental.pallas import tpu_sc as plsc

N = 10000
E = 320000
D = 128
G = 256
DEPTH = 3

NC = 2
NS = 16
NW = NC * NS
NP = 10240
CL = 128
NCH = 80
RPT = NP // NS
RCH = RPT // CL

BLK = 400
NBLK = N // BLK


def _sc_scatter_add(h, src3, dst3):
    mesh = plsc.VectorSubcoreMesh(core_axis_name="c", subcore_axis_name="s")

    @functools.partial(
        pl.kernel,
        out_type=jax.ShapeDtypeStruct((NC, NP, D), jnp.float32),
        mesh=mesh,
        scratch_types=[
            pltpu.VMEM((NCH // 2, CL), jnp.int32),
            pltpu.VMEM((NCH // 2, CL), jnp.int32),
            pltpu.VMEM((CL, D), jnp.float32),
            pltpu.VMEM((CL, D), jnp.float32),
            pltpu.VMEM_SHARED((NP, D), jnp.float32),
            pltpu.SemaphoreType.DMA,
            pltpu.SemaphoreType.DMA,
            pltpu.SemaphoreType.DMA,
            pltpu.SemaphoreType.DMA,
        ],
    )
    def k(h_hbm, src_hbm, dst_hbm, out_hbm, src_v, dst_v, buf_a, buf_b, acc,
          sem_a, sem_b, sem_sa, sem_sb):
        cid = lax.axis_index("c")
        sid = lax.axis_index("s")
        wid = sid * NC + cid
        half = NCH // 2

        pltpu.async_copy(src_hbm.at[wid, pl.ds(0, half)], src_v, sem_a)
        pltpu.async_copy(dst_hbm.at[wid, pl.ds(0, half)], dst_v, sem_b)

        def zbody(i, c):
            r = i >> 3
            col = (i & 7) << 4
            buf_a[r, pl.ds(col, 16)] = jnp.zeros((16,), jnp.float32)
            return c

        lax.fori_loop(0, CL * (D // 16), zbody, 0)
        for r in range(RCH):
            off = pl.multiple_of(sid * RPT + r * CL, CL)
            pltpu.async_copy(buf_a, acc.at[pl.ds(off, CL)], sem_sa)
        for r in range(RCH):
            off = pl.multiple_of(sid * RPT + r * CL, CL)
            pltpu.make_async_copy(buf_a, acc.at[pl.ds(off, CL)], sem_sa).wait()
        pltpu.make_async_copy(src_hbm.at[wid, pl.ds(0, half)], src_v,
                              sem_a).wait()
        pltpu.make_async_copy(dst_hbm.at[wid, pl.ds(0, half)], dst_v,
                              sem_b).wait()
        pltpu.async_copy(h_hbm.at[src_v.at[0]], buf_a, sem_a)
        plsc.subcore_barrier()

        def ebody(i, c):
            g0 = 2 * i
            g1 = g0 + 1
            pltpu.make_async_copy(h_hbm.at[src_v.at[g0]], buf_a, sem_a).wait()
            pltpu.async_copy(buf_a, acc.at[dst_v.at[g0]], sem_sa, add=True)

            @pl.when(g0 > 0)
            def _():
                pltpu.make_async_copy(buf_b, acc.at[dst_v.at[g0 - 1]],
                                      sem_sb).wait()

            pltpu.async_copy(h_hbm.at[src_v.at[g1]], buf_b, sem_b)

            pltpu.make_async_copy(h_hbm.at[src_v.at[g1]], buf_b, sem_b).wait()
            pltpu.async_copy(buf_b, acc.at[dst_v.at[g1]], sem_sb, add=True)
            pltpu.make_async_copy(buf_a, acc.at[dst_v.at[g0]], sem_sa).wait()

            @pl.when(g1 + 1 < half)
            def _():
                pltpu.async_copy(h_hbm.at[src_v.at[g1 + 1]], buf_a, sem_a)

            return c

        lax.fori_loop(0, half // 2, ebody, 0)
        pltpu.async_copy(src_hbm.at[wid, pl.ds(half, half)], src_v, sem_a)
        pltpu.make_async_copy(buf_b, acc.at[dst_v.at[half - 1]],
                              sem_sb).wait()
        pltpu.sync_copy(dst_hbm.at[wid, pl.ds(half, half)], dst_v)
        pltpu.make_async_copy(src_hbm.at[wid, pl.ds(half, half)], src_v,
                              sem_a).wait()
        pltpu.async_copy(h_hbm.at[src_v.at[0]], buf_a, sem_a)
        lax.fori_loop(0, half // 2, ebody, 0)
        pltpu.make_async_copy(buf_b, acc.at[dst_v.at[half - 1]],
                              sem_sb).wait()

        plsc.subcore_barrier()
        offs = [pl.multiple_of(sid * RPT + r * CL, CL) for r in range(RCH)]
        bufs = [buf_a if r % 2 == 0 else buf_b for r in range(RCH)]
        isem = [sem_a if r % 2 == 0 else sem_b for r in range(RCH)]
        osem = [sem_sa if r % 2 == 0 else sem_sb for r in range(RCH)]
        pltpu.async_copy(acc.at[pl.ds(offs[0], CL)], bufs[0], isem[0])
        for r in range(RCH):
            pltpu.make_async_copy(acc.at[pl.ds(offs[r], CL)], bufs[r],
                                  isem[r]).wait()
            if r >= 1:
                pltpu.make_async_copy(bufs[r - 1],
                                      out_hbm.at[cid, pl.ds(offs[r - 1], CL)],
                                      osem[r - 1]).wait()
            pltpu.async_copy(bufs[r], out_hbm.at[cid, pl.ds(offs[r], CL)],
                             osem[r])
            if r + 1 < RCH:
                pltpu.async_copy(acc.at[pl.ds(offs[r + 1], CL)], bufs[r + 1],
                                 isem[r + 1])
        pltpu.make_async_copy(bufs[RCH - 1],
                              out_hbm.at[cid, pl.ds(offs[RCH - 1], CL)],
                              osem[RCH - 1]).wait()

    return k(h, src3, dst3)


def _mlp_body(h_ref, a0_ref, a1_ref, w1_ref, b1_ref, w2_ref, b2_ref, o_ref):
    z = h_ref[...] + a0_ref[0] + a1_ref[0]
    z = jnp.maximum(
        jnp.dot(z, w1_ref[...], preferred_element_type=jnp.float32)
        + b1_ref[...], 0.0)
    o_ref[...] = jnp.maximum(
        jnp.dot(z, w2_ref[...], preferred_element_type=jnp.float32)
        + b2_ref[...], 0.0)


def _mlp(h, agg, w1f, b1f, w2, b2):
    return pl.pallas_call(
        _mlp_body,
        grid=(NBLK,),
        in_specs=[
            pl.BlockSpec((BLK, D), lambda i: (i, 0)),
            pl.BlockSpec((1, BLK, D), lambda i: (0, i, 0)),
            pl.BlockSpec((1, BLK, D), lambda i: (1, i, 0)),
            pl.BlockSpec((D, D), lambda i: (0, 0)),
            pl.BlockSpec((1, D), lambda i: (0, 0)),
            pl.BlockSpec((D, D), lambda i: (0, 0)),
            pl.BlockSpec((1, D), lambda i: (0, 0)),
        ],
        out_specs=pl.BlockSpec((BLK, D), lambda i: (i, 0)),
        out_shape=jax.ShapeDtypeStruct((N, D), jnp.float32),
    )(h, agg, agg, w1f, b1f, w2, b2)


def _mlp_pool_body(h_ref, a0_ref, a1_ref, w1_ref, b1_ref, w2_ref, b2_ref,
                   bt_ref, wp_ref, bp_ref, lg_ref, lb_ref, y_ref, seg, cnt):
    i = pl.program_id(0)

    @pl.when(i == 0)
    def _():
        seg[...] = jnp.zeros_like(seg)
        cnt[...] = jnp.zeros_like(cnt)

    z = h_ref[...] + a0_ref[0] + a1_ref[0]
    z = jnp.maximum(
        jnp.dot(z, w1_ref[...], preferred_element_type=jnp.float32)
        + b1_ref[...], 0.0)
    o = jnp.maximum(
        jnp.dot(z, w2_ref[...], preferred_element_type=jnp.float32)
        + b2_ref[...], 0.0)

    b = bt_ref[0, 0, :]
    seg_ids = lax.broadcasted_iota(jnp.int32, (G, BLK), 0)
    pf = (seg_ids == b[None, :]).astype(jnp.float32)
    seg[...] += jnp.dot(pf, o, preferred_element_type=jnp.float32)
    cnt[...] += jnp.broadcast_to(
        jnp.sum(pf, axis=1, keepdims=True), (G, D))

    @pl.when(i == NBLK - 1)
    def _():
        mean = seg[...] / jnp.maximum(cnt[...], 1.0)
        y = jnp.dot(mean, wp_ref[...],
                    preferred_element_type=jnp.float32) + bp_ref[...]
        mu = jnp.mean(y, axis=-1, keepdims=True)
        var = jnp.mean((y - mu) ** 2, axis=-1, keepdims=True)
        y_ref[...] = (y - mu) * lax.rsqrt(var + 1e-5) * lg_ref[...] + lb_ref[...]


def _mlp_pool(h, agg, w1f, b1f, w2, b2, batch3, wp, bp, ln_g, ln_b):
    return pl.pallas_call(
        _mlp_pool_body,
        grid=(NBLK,),
        in_specs=[
            pl.BlockSpec((BLK, D), lambda i: (i, 0)),
            pl.BlockSpec((1, BLK, D), lambda i: (0, i, 0)),
            pl.BlockSpec((1, BLK, D), lambda i: (1, i, 0)),
            pl.BlockSpec((D, D), lambda i: (0, 0)),
            pl.BlockSpec((1, D), lambda i: (0, 0)),
            pl.BlockSpec((D, D), lambda i: (0, 0)),
            pl.BlockSpec((1, D), lambda i: (0, 0)),
            pl.BlockSpec((1, 1, BLK), lambda i: (i, 0, 0)),
            pl.BlockSpec((D, D), lambda i: (0, 0)),
            pl.BlockSpec((1, D), lambda i: (0, 0)),
            pl.BlockSpec((1, D), lambda i: (0, 0)),
            pl.BlockSpec((1, D), lambda i: (0, 0)),
        ],
        out_specs=pl.BlockSpec((G, D), lambda i: (0, 0)),
        out_shape=jax.ShapeDtypeStruct((G, D), jnp.float32),
        scratch_shapes=[
            pltpu.VMEM((G, D), jnp.float32),
            pltpu.VMEM((G, D), jnp.float32),
        ],
    )(h, agg, agg, w1f, b1f, w2, b2, batch3, wp, bp, ln_g, ln_b)


def kernel(x, edge_index, batch, w1, b1, bn_g, bn_b, bn_rm, bn_rv, w2, b2,
           wp, bp, ln_g, ln_b):
    scale = bn_g / jnp.sqrt(bn_rv + 1e-5)
    w1f = w1 * scale[:, None, :]
    b1f = (b1 - bn_rm) * scale + bn_b

    pad = NW * NCH * CL - E
    pad_ar = jnp.arange(pad, dtype=jnp.int32)
    src3 = jnp.concatenate(
        [edge_index[0], pad_ar % N]).reshape(NW, NCH, CL)
    dst3 = jnp.concatenate(
        [edge_index[1], N + pad_ar % (NP - N)]).reshape(NW, NCH, CL)
    batch3 = batch.reshape(NBLK, 1, BLK)

    h = x
    for i in range(DEPTH - 1):
        agg = _sc_scatter_add(h, src3, dst3)
        h = _mlp(h, agg, w1f[i], b1f[i][None], w2[i], b2[i][None])
    agg = _sc_scatter_add(h, src3, dst3)
    return _mlp_pool(h, agg, w1f[2], b1f[2][None], w2[2], b2[2][None],
                     batch3, wp, bp[None], ln_g[None], ln_b[None])


# speedup vs baseline: 10.2987x; 1.0765x over previous
import functools

import jax
import jax.numpy as jnp
from jax import lax
from jax.experimental import pallas as pl
from jax.experimental.pallas import tpu as pltpu
from jax.experimental.pallas import tpu_sc as plsc

N = 10000
E = 320000
D = 128
G = 256
DEPTH = 3

NC = 2
NS = 16
NW = NC * NS
NP = 10240
CL = 128
NCH = 80
RPT = NP // NS
RCH = RPT // CL

BLK = 2000
NBLK = N // BLK


def _sc_scatter_add(h, src3, dst3):
    mesh = plsc.VectorSubcoreMesh(core_axis_name="c", subcore_axis_name="s")

    @functools.partial(
        pl.kernel,
        out_type=jax.ShapeDtypeStruct((NC, NP, D), jnp.float32),
        mesh=mesh,
        scratch_types=[
            pltpu.VMEM((NCH // 2, CL), jnp.int32),
            pltpu.VMEM((NCH // 2, CL), jnp.int32),
            pltpu.VMEM((CL, D), jnp.float32),
            pltpu.VMEM((CL, D), jnp.float32),
            pltpu.VMEM_SHARED((NP, D), jnp.float32),
            pltpu.SemaphoreType.DMA,
            pltpu.SemaphoreType.DMA,
            pltpu.SemaphoreType.DMA,
            pltpu.SemaphoreType.DMA,
        ],
    )
    def k(h_hbm, src_hbm, dst_hbm, out_hbm, src_v, dst_v, buf_a, buf_b, acc,
          sem_a, sem_b, sem_sa, sem_sb):
        cid = lax.axis_index("c")
        sid = lax.axis_index("s")
        wid = sid * NC + cid
        half = NCH // 2

        pltpu.async_copy(src_hbm.at[wid, pl.ds(0, half)], src_v, sem_a)
        pltpu.async_copy(dst_hbm.at[wid, pl.ds(0, half)], dst_v, sem_b)

        def zbody(i, c):
            r = i >> 3
            col = (i & 7) << 4
            buf_a[r, pl.ds(col, 16)] = jnp.zeros((16,), jnp.float32)
            return c

        lax.fori_loop(0, CL * (D // 16), zbody, 0)
        for r in range(RCH):
            off = pl.multiple_of(sid * RPT + r * CL, CL)
            pltpu.async_copy(buf_a, acc.at[pl.ds(off, CL)], sem_sa)
        for r in range(RCH):
            off = pl.multiple_of(sid * RPT + r * CL, CL)
            pltpu.make_async_copy(buf_a, acc.at[pl.ds(off, CL)], sem_sa).wait()
        pltpu.make_async_copy(src_hbm.at[wid, pl.ds(0, half)], src_v,
                              sem_a).wait()
        pltpu.make_async_copy(dst_hbm.at[wid, pl.ds(0, half)], dst_v,
                              sem_b).wait()
        pltpu.async_copy(h_hbm.at[src_v.at[0]], buf_a, sem_a)
        plsc.subcore_barrier()

        def ebody(i, c):
            g0 = 2 * i
            g1 = g0 + 1
            pltpu.make_async_copy(h_hbm.at[src_v.at[g0]], buf_a, sem_a).wait()
            pltpu.async_copy(buf_a, acc.at[dst_v.at[g0]], sem_sa, add=True)

            @pl.when(g0 > 0)
            def _():
                pltpu.make_async_copy(buf_b, acc.at[dst_v.at[g0 - 1]],
                                      sem_sb).wait()

            pltpu.async_copy(h_hbm.at[src_v.at[g1]], buf_b, sem_b)

            pltpu.make_async_copy(h_hbm.at[src_v.at[g1]], buf_b, sem_b).wait()
            pltpu.async_copy(buf_b, acc.at[dst_v.at[g1]], sem_sb, add=True)
            pltpu.make_async_copy(buf_a, acc.at[dst_v.at[g0]], sem_sa).wait()

            @pl.when(g1 + 1 < half)
            def _():
                pltpu.async_copy(h_hbm.at[src_v.at[g1 + 1]], buf_a, sem_a)

            return c

        lax.fori_loop(0, half // 2, ebody, 0)
        pltpu.async_copy(src_hbm.at[wid, pl.ds(half, half)], src_v, sem_a)
        pltpu.make_async_copy(buf_b, acc.at[dst_v.at[half - 1]],
                              sem_sb).wait()
        pltpu.sync_copy(dst_hbm.at[wid, pl.ds(half, half)], dst_v)
        pltpu.make_async_copy(src_hbm.at[wid, pl.ds(half, half)], src_v,
                              sem_a).wait()
        pltpu.async_copy(h_hbm.at[src_v.at[0]], buf_a, sem_a)
        lax.fori_loop(0, half // 2, ebody, 0)
        pltpu.make_async_copy(buf_b, acc.at[dst_v.at[half - 1]],
                              sem_sb).wait()

        plsc.subcore_barrier()
        offs = [pl.multiple_of(sid * RPT + r * CL, CL) for r in range(RCH)]
        bufs = [buf_a if r % 2 == 0 else buf_b for r in range(RCH)]
        isem = [sem_a if r % 2 == 0 else sem_b for r in range(RCH)]
        osem = [sem_sa if r % 2 == 0 else sem_sb for r in range(RCH)]
        pltpu.async_copy(acc.at[pl.ds(offs[0], CL)], bufs[0], isem[0])
        for r in range(RCH):
            pltpu.make_async_copy(acc.at[pl.ds(offs[r], CL)], bufs[r],
                                  isem[r]).wait()
            if r >= 1:
                pltpu.make_async_copy(bufs[r - 1],
                                      out_hbm.at[cid, pl.ds(offs[r - 1], CL)],
                                      osem[r - 1]).wait()
            pltpu.async_copy(bufs[r], out_hbm.at[cid, pl.ds(offs[r], CL)],
                             osem[r])
            if r + 1 < RCH:
                pltpu.async_copy(acc.at[pl.ds(offs[r + 1], CL)], bufs[r + 1],
                                 isem[r + 1])
        pltpu.make_async_copy(bufs[RCH - 1],
                              out_hbm.at[cid, pl.ds(offs[RCH - 1], CL)],
                              osem[RCH - 1]).wait()

    return k(h, src3, dst3)


def _mlp_body(h_ref, a0_ref, a1_ref, w1_ref, b1_ref, w2_ref, b2_ref, o_ref):
    z = h_ref[...] + a0_ref[0] + a1_ref[0]
    z = jnp.maximum(
        jnp.dot(z, w1_ref[...], preferred_element_type=jnp.float32)
        + b1_ref[...], 0.0)
    o_ref[...] = jnp.maximum(
        jnp.dot(z, w2_ref[...], preferred_element_type=jnp.float32)
        + b2_ref[...], 0.0)


def _mlp(h, agg, w1f, b1f, w2, b2):
    return pl.pallas_call(
        _mlp_body,
        grid=(NBLK,),
        in_specs=[
            pl.BlockSpec((BLK, D), lambda i: (i, 0)),
            pl.BlockSpec((1, BLK, D), lambda i: (0, i, 0)),
            pl.BlockSpec((1, BLK, D), lambda i: (1, i, 0)),
            pl.BlockSpec((D, D), lambda i: (0, 0)),
            pl.BlockSpec((1, D), lambda i: (0, 0)),
            pl.BlockSpec((D, D), lambda i: (0, 0)),
            pl.BlockSpec((1, D), lambda i: (0, 0)),
        ],
        out_specs=pl.BlockSpec((BLK, D), lambda i: (i, 0)),
        out_shape=jax.ShapeDtypeStruct((N, D), jnp.float32),
    )(h, agg, agg, w1f, b1f, w2, b2)


def _mlp_pool_body(h_ref, a0_ref, a1_ref, w1_ref, b1_ref, w2_ref, b2_ref,
                   bt_ref, wp_ref, bp_ref, lg_ref, lb_ref, y_ref, seg, cnt):
    i = pl.program_id(0)

    @pl.when(i == 0)
    def _():
        seg[...] = jnp.zeros_like(seg)
        cnt[...] = jnp.zeros_like(cnt)

    z = h_ref[...] + a0_ref[0] + a1_ref[0]
    z = jnp.maximum(
        jnp.dot(z, w1_ref[...], preferred_element_type=jnp.float32)
        + b1_ref[...], 0.0)
    o = jnp.maximum(
        jnp.dot(z, w2_ref[...], preferred_element_type=jnp.float32)
        + b2_ref[...], 0.0)

    b = bt_ref[0, 0, :]
    seg_ids = lax.broadcasted_iota(jnp.int32, (G, BLK), 0)
    pf = (seg_ids == b[None, :]).astype(jnp.float32)
    seg[...] += jnp.dot(pf, o, preferred_element_type=jnp.float32)
    cnt[...] += jnp.broadcast_to(
        jnp.sum(pf, axis=1, keepdims=True), (G, D))

    @pl.when(i == NBLK - 1)
    def _():
        mean = seg[...] / jnp.maximum(cnt[...], 1.0)
        y = jnp.dot(mean, wp_ref[...],
                    preferred_element_type=jnp.float32) + bp_ref[...]
        mu = jnp.mean(y, axis=-1, keepdims=True)
        var = jnp.mean((y - mu) ** 2, axis=-1, keepdims=True)
        y_ref[...] = (y - mu) * lax.rsqrt(var + 1e-5) * lg_ref[...] + lb_ref[...]


def _mlp_pool(h, agg, w1f, b1f, w2, b2, batch3, wp, bp, ln_g, ln_b):
    return pl.pallas_call(
        _mlp_pool_body,
        grid=(NBLK,),
        in_specs=[
            pl.BlockSpec((BLK, D), lambda i: (i, 0)),
            pl.BlockSpec((1, BLK, D), lambda i: (0, i, 0)),
            pl.BlockSpec((1, BLK, D), lambda i: (1, i, 0)),
            pl.BlockSpec((D, D), lambda i: (0, 0)),
            pl.BlockSpec((1, D), lambda i: (0, 0)),
            pl.BlockSpec((D, D), lambda i: (0, 0)),
            pl.BlockSpec((1, D), lambda i: (0, 0)),
            pl.BlockSpec((1, 1, BLK), lambda i: (i, 0, 0)),
            pl.BlockSpec((D, D), lambda i: (0, 0)),
            pl.BlockSpec((1, D), lambda i: (0, 0)),
            pl.BlockSpec((1, D), lambda i: (0, 0)),
            pl.BlockSpec((1, D), lambda i: (0, 0)),
        ],
        out_specs=pl.BlockSpec((G, D), lambda i: (0, 0)),
        out_shape=jax.ShapeDtypeStruct((G, D), jnp.float32),
        scratch_shapes=[
            pltpu.VMEM((G, D), jnp.float32),
            pltpu.VMEM((G, D), jnp.float32),
        ],
    )(h, agg, agg, w1f, b1f, w2, b2, batch3, wp, bp, ln_g, ln_b)


def kernel(x, edge_index, batch, w1, b1, bn_g, bn_b, bn_rm, bn_rv, w2, b2,
           wp, bp, ln_g, ln_b):
    scale = bn_g / jnp.sqrt(bn_rv + 1e-5)
    w1f = w1 * scale[:, None, :]
    b1f = (b1 - bn_rm) * scale + bn_b

    pad = NW * NCH * CL - E
    pad_ar = jnp.arange(pad, dtype=jnp.int32)
    src3 = jnp.concatenate(
        [edge_index[0], pad_ar % N]).reshape(NW, NCH, CL)
    dst3 = jnp.concatenate(
        [edge_index[1], N + pad_ar % (NP - N)]).reshape(NW, NCH, CL)
    batch3 = batch.reshape(NBLK, 1, BLK)

    h = x
    for i in range(DEPTH - 1):
        agg = _sc_scatter_add(h, src3, dst3)
        h = _mlp(h, agg, w1f[i], b1f[i][None], w2[i], b2[i][None])
    agg = _sc_scatter_add(h, src3, dst3)
    return _mlp_pool(h, agg, w1f[2], b1f[2][None], w2[2], b2[2][None],
                     batch3, wp, bp[None], ln_g[None], ln_b[None])


# speedup vs baseline: 10.3528x; 1.0053x over previous
import functools

import jax
import jax.numpy as jnp
from jax import lax
from jax.experimental import pallas as pl
from jax.experimental.pallas import tpu as pltpu
from jax.experimental.pallas import tpu_sc as plsc

N = 10000
E = 320000
D = 128
G = 256
DEPTH = 3

NC = 2
NS = 16
NW = NC * NS
NP = 10240
CL = 128
NCH = 80
RPT = NP // NS
RCH = RPT // CL

BLK = 5000
NBLK = N // BLK


def _sc_scatter_add(h, src3, dst3):
    mesh = plsc.VectorSubcoreMesh(core_axis_name="c", subcore_axis_name="s")

    @functools.partial(
        pl.kernel,
        out_type=jax.ShapeDtypeStruct((NC, NP, D), jnp.float32),
        mesh=mesh,
        scratch_types=[
            pltpu.VMEM((NCH // 2, CL), jnp.int32),
            pltpu.VMEM((NCH // 2, CL), jnp.int32),
            pltpu.VMEM((CL, D), jnp.float32),
            pltpu.VMEM((CL, D), jnp.float32),
            pltpu.VMEM_SHARED((NP, D), jnp.float32),
            pltpu.SemaphoreType.DMA,
            pltpu.SemaphoreType.DMA,
            pltpu.SemaphoreType.DMA,
            pltpu.SemaphoreType.DMA,
        ],
    )
    def k(h_hbm, src_hbm, dst_hbm, out_hbm, src_v, dst_v, buf_a, buf_b, acc,
          sem_a, sem_b, sem_sa, sem_sb):
        cid = lax.axis_index("c")
        sid = lax.axis_index("s")
        wid = sid * NC + cid
        half = NCH // 2

        pltpu.async_copy(src_hbm.at[wid, pl.ds(0, half)], src_v, sem_a)
        pltpu.async_copy(dst_hbm.at[wid, pl.ds(0, half)], dst_v, sem_b)

        def zbody(i, c):
            r = i >> 3
            col = (i & 7) << 4
            buf_a[r, pl.ds(col, 16)] = jnp.zeros((16,), jnp.float32)
            return c

        lax.fori_loop(0, CL * (D // 16), zbody, 0)
        for r in range(RCH):
            off = pl.multiple_of(sid * RPT + r * CL, CL)
            pltpu.async_copy(buf_a, acc.at[pl.ds(off, CL)], sem_sa)
        for r in range(RCH):
            off = pl.multiple_of(sid * RPT + r * CL, CL)
            pltpu.make_async_copy(buf_a, acc.at[pl.ds(off, CL)], sem_sa).wait()
        pltpu.make_async_copy(src_hbm.at[wid, pl.ds(0, half)], src_v,
                              sem_a).wait()
        pltpu.make_async_copy(dst_hbm.at[wid, pl.ds(0, half)], dst_v,
                              sem_b).wait()
        pltpu.async_copy(h_hbm.at[src_v.at[0]], buf_a, sem_a)
        plsc.subcore_barrier()

        def ebody(i, c):
            g0 = 2 * i
            g1 = g0 + 1
            pltpu.make_async_copy(h_hbm.at[src_v.at[g0]], buf_a, sem_a).wait()
            pltpu.async_copy(buf_a, acc.at[dst_v.at[g0]], sem_sa, add=True)

            @pl.when(g0 > 0)
            def _():
                pltpu.make_async_copy(buf_b, acc.at[dst_v.at[g0 - 1]],
                                      sem_sb).wait()

            pltpu.async_copy(h_hbm.at[src_v.at[g1]], buf_b, sem_b)

            pltpu.make_async_copy(h_hbm.at[src_v.at[g1]], buf_b, sem_b).wait()
            pltpu.async_copy(buf_b, acc.at[dst_v.at[g1]], sem_sb, add=True)
            pltpu.make_async_copy(buf_a, acc.at[dst_v.at[g0]], sem_sa).wait()

            @pl.when(g1 + 1 < half)
            def _():
                pltpu.async_copy(h_hbm.at[src_v.at[g1 + 1]], buf_a, sem_a)

            return c

        lax.fori_loop(0, half // 2, ebody, 0)
        pltpu.async_copy(src_hbm.at[wid, pl.ds(half, half)], src_v, sem_a)
        pltpu.make_async_copy(buf_b, acc.at[dst_v.at[half - 1]],
                              sem_sb).wait()
        pltpu.sync_copy(dst_hbm.at[wid, pl.ds(half, half)], dst_v)
        pltpu.make_async_copy(src_hbm.at[wid, pl.ds(half, half)], src_v,
                              sem_a).wait()
        pltpu.async_copy(h_hbm.at[src_v.at[0]], buf_a, sem_a)
        lax.fori_loop(0, half // 2, ebody, 0)
        pltpu.make_async_copy(buf_b, acc.at[dst_v.at[half - 1]],
                              sem_sb).wait()

        plsc.subcore_barrier()
        offs = [pl.multiple_of(sid * RPT + r * CL, CL) for r in range(RCH)]
        bufs = [buf_a if r % 2 == 0 else buf_b for r in range(RCH)]
        isem = [sem_a if r % 2 == 0 else sem_b for r in range(RCH)]
        osem = [sem_sa if r % 2 == 0 else sem_sb for r in range(RCH)]
        pltpu.async_copy(acc.at[pl.ds(offs[0], CL)], bufs[0], isem[0])
        for r in range(RCH):
            pltpu.make_async_copy(acc.at[pl.ds(offs[r], CL)], bufs[r],
                                  isem[r]).wait()
            if r >= 1:
                pltpu.make_async_copy(bufs[r - 1],
                                      out_hbm.at[cid, pl.ds(offs[r - 1], CL)],
                                      osem[r - 1]).wait()
            pltpu.async_copy(bufs[r], out_hbm.at[cid, pl.ds(offs[r], CL)],
                             osem[r])
            if r + 1 < RCH:
                pltpu.async_copy(acc.at[pl.ds(offs[r + 1], CL)], bufs[r + 1],
                                 isem[r + 1])
        pltpu.make_async_copy(bufs[RCH - 1],
                              out_hbm.at[cid, pl.ds(offs[RCH - 1], CL)],
                              osem[RCH - 1]).wait()

    return k(h, src3, dst3)


def _mlp_body(h_ref, a0_ref, a1_ref, w1_ref, b1_ref, w2_ref, b2_ref, o_ref):
    z = h_ref[...] + a0_ref[0] + a1_ref[0]
    z = jnp.maximum(
        jnp.dot(z, w1_ref[...], preferred_element_type=jnp.float32)
        + b1_ref[...], 0.0)
    o_ref[...] = jnp.maximum(
        jnp.dot(z, w2_ref[...], preferred_element_type=jnp.float32)
        + b2_ref[...], 0.0)


def _mlp(h, agg, w1f, b1f, w2, b2):
    return pl.pallas_call(
        _mlp_body,
        grid=(NBLK,),
        in_specs=[
            pl.BlockSpec((BLK, D), lambda i: (i, 0)),
            pl.BlockSpec((1, BLK, D), lambda i: (0, i, 0)),
            pl.BlockSpec((1, BLK, D), lambda i: (1, i, 0)),
            pl.BlockSpec((D, D), lambda i: (0, 0)),
            pl.BlockSpec((1, D), lambda i: (0, 0)),
            pl.BlockSpec((D, D), lambda i: (0, 0)),
            pl.BlockSpec((1, D), lambda i: (0, 0)),
        ],
        out_specs=pl.BlockSpec((BLK, D), lambda i: (i, 0)),
        out_shape=jax.ShapeDtypeStruct((N, D), jnp.float32),
    )(h, agg, agg, w1f, b1f, w2, b2)


def _mlp_pool_body(h_ref, a0_ref, a1_ref, w1_ref, b1_ref, w2_ref, b2_ref,
                   bt_ref, wp_ref, bp_ref, lg_ref, lb_ref, y_ref, seg, cnt):
    i = pl.program_id(0)

    @pl.when(i == 0)
    def _():
        seg[...] = jnp.zeros_like(seg)
        cnt[...] = jnp.zeros_like(cnt)

    z = h_ref[...] + a0_ref[0] + a1_ref[0]
    z = jnp.maximum(
        jnp.dot(z, w1_ref[...], preferred_element_type=jnp.float32)
        + b1_ref[...], 0.0)
    o = jnp.maximum(
        jnp.dot(z, w2_ref[...], preferred_element_type=jnp.float32)
        + b2_ref[...], 0.0)

    b = bt_ref[0, 0, :]
    seg_ids = lax.broadcasted_iota(jnp.int32, (G, BLK), 0)
    pf = (seg_ids == b[None, :]).astype(jnp.float32)
    seg[...] += jnp.dot(pf, o, preferred_element_type=jnp.float32)
    cnt[...] += jnp.broadcast_to(
        jnp.sum(pf, axis=1, keepdims=True), (G, D))

    @pl.when(i == NBLK - 1)
    def _():
        mean = seg[...] / jnp.maximum(cnt[...], 1.0)
        y = jnp.dot(mean, wp_ref[...],
                    preferred_element_type=jnp.float32) + bp_ref[...]
        mu = jnp.mean(y, axis=-1, keepdims=True)
        var = jnp.mean((y - mu) ** 2, axis=-1, keepdims=True)
        y_ref[...] = (y - mu) * lax.rsqrt(var + 1e-5) * lg_ref[...] + lb_ref[...]


def _mlp_pool(h, agg, w1f, b1f, w2, b2, batch3, wp, bp, ln_g, ln_b):
    return pl.pallas_call(
        _mlp_pool_body,
        grid=(NBLK,),
        in_specs=[
            pl.BlockSpec((BLK, D), lambda i: (i, 0)),
            pl.BlockSpec((1, BLK, D), lambda i: (0, i, 0)),
            pl.BlockSpec((1, BLK, D), lambda i: (1, i, 0)),
            pl.BlockSpec((D, D), lambda i: (0, 0)),
            pl.BlockSpec((1, D), lambda i: (0, 0)),
            pl.BlockSpec((D, D), lambda i: (0, 0)),
            pl.BlockSpec((1, D), lambda i: (0, 0)),
            pl.BlockSpec((1, 1, BLK), lambda i: (i, 0, 0)),
            pl.BlockSpec((D, D), lambda i: (0, 0)),
            pl.BlockSpec((1, D), lambda i: (0, 0)),
            pl.BlockSpec((1, D), lambda i: (0, 0)),
            pl.BlockSpec((1, D), lambda i: (0, 0)),
        ],
        out_specs=pl.BlockSpec((G, D), lambda i: (0, 0)),
        out_shape=jax.ShapeDtypeStruct((G, D), jnp.float32),
        scratch_shapes=[
            pltpu.VMEM((G, D), jnp.float32),
            pltpu.VMEM((G, D), jnp.float32),
        ],
    )(h, agg, agg, w1f, b1f, w2, b2, batch3, wp, bp, ln_g, ln_b)


def kernel(x, edge_index, batch, w1, b1, bn_g, bn_b, bn_rm, bn_rv, w2, b2,
           wp, bp, ln_g, ln_b):
    scale = bn_g / jnp.sqrt(bn_rv + 1e-5)
    w1f = w1 * scale[:, None, :]
    b1f = (b1 - bn_rm) * scale + bn_b

    pad = NW * NCH * CL - E
    pad_ar = jnp.arange(pad, dtype=jnp.int32)
    src3 = jnp.concatenate(
        [edge_index[0], pad_ar % N]).reshape(NW, NCH, CL)
    dst3 = jnp.concatenate(
        [edge_index[1], N + pad_ar % (NP - N)]).reshape(NW, NCH, CL)
    batch3 = batch.reshape(NBLK, 1, BLK)

    h = x
    for i in range(DEPTH - 1):
        agg = _sc_scatter_add(h, src3, dst3)
        h = _mlp(h, agg, w1f[i], b1f[i][None], w2[i], b2[i][None])
    agg = _sc_scatter_add(h, src3, dst3)
    return _mlp_pool(h, agg, w1f[2], b1f[2][None], w2[2], b2[2][None],
                     batch3, wp, bp[None], ln_g[None], ln_b[None])
